# baseline (speedup 1.0000x reference)
import functools

import jax
import jax.numpy as jnp
from jax import lax
from jax.experimental import pallas as pl
from jax.experimental.pallas import tpu as pltpu
from jax.experimental.pallas import tpu_sc as plsc

_H, _W, _C = 512, 512, 3
_N = _H * _W * _C
_NC, _NS = 2, 16
_NW = _NC * _NS
_CHUNK = _N // _NW
_Z_ROW0 = (10 * _W + 10) * _C
_Z_ROW1 = (11 * _W + 10) * _C


def _sc_body(img_hbm, out_hbm, buf):
    wid = lax.axis_index("s") * _NC + lax.axis_index("c")
    base = wid * _CHUNK
    pltpu.sync_copy(img_hbm.at[pl.ds(base, _CHUNK)], buf)

    @pl.when(wid == 0)
    def _zero_pixels():
        lanes = lax.iota(jnp.int32, 16)
        keep = (lanes < 6) | (lanes >= 12)
        for b in (_Z_ROW0 - 6, _Z_ROW1 - 6):
            v = buf[pl.ds(b, 16)]
            buf[pl.ds(b, 16)] = jnp.where(keep, v, 0.0)

    pltpu.sync_copy(buf, out_hbm.at[pl.ds(base, _CHUNK)])


@jax.jit
def kernel(img):
    mesh = plsc.VectorSubcoreMesh(core_axis_name="c", subcore_axis_name="s")
    run = functools.partial(
        pl.kernel,
        mesh=mesh,
        out_type=jax.ShapeDtypeStruct((_N,), jnp.float32),
        scratch_types=[pltpu.VMEM((_CHUNK,), jnp.float32)],
    )(_sc_body)
    out = run(img.reshape(_N))
    return out.reshape(_H, _W, _C)


# speedup vs baseline: 1.0004x; 1.0004x over previous
import functools

import jax
import jax.numpy as jnp
from jax import lax
from jax.experimental import pallas as pl
from jax.experimental.pallas import tpu as pltpu
from jax.experimental.pallas import tpu_sc as plsc

_H, _W, _C = 512, 512, 3
_N = _H * _W * _C


def _sc_body(img_hbm, out_hbm, buf):
    wid = lax.axis_index("s") * 2 + lax.axis_index("c")

    @pl.when(wid == 0)
    def _():
        pltpu.sync_copy(img_hbm.at[pl.ds(0, 16)], buf)
        pltpu.sync_copy(buf, out_hbm.at[pl.ds(0, 16)])


@jax.jit
def kernel(img):
    mesh = plsc.VectorSubcoreMesh(core_axis_name="c", subcore_axis_name="s")
    run = functools.partial(
        pl.kernel,
        mesh=mesh,
        out_type=jax.ShapeDtypeStruct((_N,), jnp.float32),
        scratch_types=[pltpu.VMEM((16,), jnp.float32)],
    )(_sc_body)
    out = run(img.reshape(_N))
    return out.reshape(_H, _W, _C)


# speedup vs baseline: 13.5640x; 13.5590x over previous
import jax
import jax.numpy as jnp
from jax import lax
from jax.experimental import pallas as pl

_H, _W, _C = 512, 512, 3
_WC = _W * _C
_BLK_ROWS = 64
_COL0 = 10 * _C


def _body(in_ref, out_ref):
    out_ref[...] = in_ref[...]

    @pl.when(pl.program_id(0) == 0)
    def _zero_pixels():
        sub = out_ref[8:16, 0:128]
        r = lax.broadcasted_iota(jnp.int32, (8, 128), 0)
        c = lax.broadcasted_iota(jnp.int32, (8, 128), 1)
        hit = ((r == 2) | (r == 3)) & (c >= _COL0) & (c < _COL0 + 2 * _C)
        out_ref[8:16, 0:128] = jnp.where(hit, 0.0, sub)


@jax.jit
def kernel(img):
    x = img.reshape(_H, _WC)
    out = pl.pallas_call(
        _body,
        grid=(_H // _BLK_ROWS,),
        in_specs=[pl.BlockSpec((_BLK_ROWS, _WC), lambda i: (i, 0))],
        out_specs=pl.BlockSpec((_BLK_ROWS, _WC), lambda i: (i, 0)),
        out_shape=jax.ShapeDtypeStruct((_H, _WC), jnp.float32),
    )(x)
    return out.reshape(_H, _W, _C)
